# baseline (speedup 1.0000x reference)
import functools

import jax
import jax.numpy as jnp
from jax import lax
from jax.experimental import pallas as pl
from jax.experimental.pallas import tpu as pltpu
from jax.experimental.pallas import tpu_sc as plsc

D = 32
NC = 2
NS = 16
NW = NC * NS
B = 16384
H = 200
BT = B // 128
BT_PER_W = BT // NW
HG = H // 8
NSLOT = HG * BT_PER_W


def _make_lookup():
    mesh = plsc.VectorSubcoreMesh(core_axis_name="c", subcore_axis_name="s")

    @functools.partial(
        pl.kernel,
        out_type=jax.ShapeDtypeStruct((H, D // 8, BT, 8, 128), jnp.float32),
        mesh=mesh,
        scratch_types=[
            pltpu.VMEM((2, BT_PER_W, 8, 128), jnp.int32),
            pltpu.VMEM((2048, D), jnp.float32),
            pltpu.VMEM((8, D // 8, 8, 128), jnp.float32),
            pltpu.SemaphoreType.DMA,
            pltpu.SemaphoreType.DMA,
            pltpu.SemaphoreType.DMA,
        ],
        compiler_params=pltpu.CompilerParams(
            use_tc_tiling_on_sc=False, needs_layout_passes=False
        ),
    )
    def lookup(idx_hbm, table_hbm, out_hbm, idx_v, gv, tv, gsem, osem, isem):
        wid = lax.axis_index("s") * NC + lax.axis_index("c")
        bt0 = wid * BT_PER_W
        iota = jnp.arange(16, dtype=jnp.int32)

        def fire_slab(g, q):
            pltpu.async_copy(
                idx_hbm.at[g, pl.ds(bt0, BT_PER_W)], idx_v.at[q], isem
            )

        def drain_slab():
            pltpu.make_async_copy(
                idx_hbm.at[0, pl.ds(0, BT_PER_W)], idx_v.at[0], isem
            ).wait()

        def fire_gathers(s):
            g = s >> 2
            btl = s & 3
            q = g & 1
            p = s & 1
            for i in range(8):
                pltpu.async_copy(
                    table_hbm.at[idx_v.at[q, btl, i]],
                    gv.at[pl.ds(p * 1024 + i * 128, 128)],
                    gsem,
                )

        def drain_gathers():
            for i in range(8):
                pltpu.make_async_copy(
                    table_hbm.at[pl.ds(0, 128)],
                    gv.at[pl.ds(i * 128, 128)],
                    gsem,
                ).wait()

        def drain_stores():
            for b in range(8):
                pltpu.make_async_copy(
                    tv.at[b], out_hbm.at[0, :, 0], osem
                ).wait()

        def process(s):
            g = s >> 2
            btl = s & 3
            p = s & 1
            base0 = p * 1024

            @plsc.parallel_loop(0, 8, unroll=2)
            def tblk(blk):
                base = base0 + blk * 128
                rows = [iota + (base + j16 * 16) for j16 in range(8)]
                for d in range(D):
                    cold = jnp.full((16,), d, dtype=jnp.int32)
                    for j16 in range(8):
                        v = plsc.load_gather(gv, [rows[j16], cold])
                        tv[blk, d // 8, d % 8, pl.ds(j16 * 16, 16)] = v

            for b in range(8):
                pltpu.async_copy(
                    tv.at[b], out_hbm.at[8 * g + b, :, bt0 + btl], osem
                )

        pltpu.sync_copy(idx_hbm.at[0, pl.ds(bt0, BT_PER_W)], idx_v.at[0])
        fire_gathers(0)
        fire_slab(1, 1)

        def body(gi, carry):
            s = gi + 1

            @pl.when((s & 3) == 0)
            def _():
                drain_slab()

            fire_gathers(s)
            drain_gathers()

            @pl.when(((s & 3) == 0) & (s <= 4 * (HG - 1) - 4))
            def _():
                fire_slab((s >> 2) + 1, ((s >> 2) + 1) & 1)

            @pl.when(gi > 0)
            def _():
                drain_stores()

            process(gi)
            return carry

        lax.fori_loop(0, NSLOT - 1, body, 0)
        drain_stores()
        drain_gathers()
        process(NSLOT - 1)
        drain_stores()

    return lookup


_LOOKUP = _make_lookup()


def kernel(inputs, embeddings):
    idx_p = inputs.T.reshape(HG, 8, BT, 128).transpose(0, 2, 1, 3)
    out_p = _LOOKUP(idx_p, embeddings)
    x = out_p.transpose(0, 1, 3, 2, 4).reshape(H, D, B)
    return x.transpose(2, 0, 1)


# speedup vs baseline: 1.2846x; 1.2846x over previous
import functools

import jax
import jax.numpy as jnp
from jax import lax
from jax.experimental import pallas as pl
from jax.experimental.pallas import tpu as pltpu
from jax.experimental.pallas import tpu_sc as plsc

D = 32
NC = 2
NS = 16
NW = NC * NS
B = 16384
H = 200
BT = B // 128
BT_PER_W = BT // NW
HG = H // 8
NSLOT = HG * BT_PER_W


def _make_lookup():
    mesh = plsc.VectorSubcoreMesh(core_axis_name="c", subcore_axis_name="s")

    @functools.partial(
        pl.kernel,
        out_type=jax.ShapeDtypeStruct((H, D // 8, BT, 8, 128), jnp.float32),
        mesh=mesh,
        scratch_types=[
            pltpu.VMEM((2, BT_PER_W, 8, 128), jnp.int32),
            pltpu.VMEM((2048, D), jnp.float32),
            pltpu.VMEM((8, D // 8, 8, 128), jnp.float32),
            pltpu.SemaphoreType.DMA,
            pltpu.SemaphoreType.DMA,
            pltpu.SemaphoreType.DMA,
        ],
        compiler_params=pltpu.CompilerParams(
            use_tc_tiling_on_sc=False, needs_layout_passes=False
        ),
    )
    def lookup(idx_hbm, table_hbm, out_hbm, idx_v, gv, tv, gsem, osem, isem):
        wid = lax.axis_index("s") * NC + lax.axis_index("c")
        bt0 = wid * BT_PER_W
        iota = jnp.arange(16, dtype=jnp.int32)

        def fire_slab(g, q):
            pltpu.async_copy(
                idx_hbm.at[g, pl.ds(bt0, BT_PER_W)], idx_v.at[q], isem
            )

        def drain_slab():
            pltpu.make_async_copy(
                idx_hbm.at[0, pl.ds(0, BT_PER_W)], idx_v.at[0], isem
            ).wait()

        def fire_gathers(s):
            g = s >> 2
            btl = s & 3
            q = g & 1
            p = s & 1
            for i in range(8):
                pltpu.async_copy(
                    table_hbm.at[idx_v.at[q, btl, i]],
                    gv.at[pl.ds(p * 1024 + i * 128, 128)],
                    gsem,
                )

        def drain_gathers():
            for i in range(8):
                pltpu.make_async_copy(
                    table_hbm.at[pl.ds(0, 128)],
                    gv.at[pl.ds(i * 128, 128)],
                    gsem,
                ).wait()

        def drain_stores():
            for b in range(8):
                pltpu.make_async_copy(
                    tv.at[b], out_hbm.at[0, :, 0], osem
                ).wait()

        def process(s):
            g = s >> 2
            btl = s & 3
            p = s & 1
            base0 = p * 1024

            @plsc.parallel_loop(0, 8, unroll=2)
            def tblk(blk):
                base = base0 + blk * 128
                rows = [iota + (base + j16 * 16) for j16 in range(8)]
                for d0 in range(0, D, 2):
                    vs = []
                    for d in (d0, d0 + 1):
                        cold = jnp.full((16,), d, dtype=jnp.int32)
                        for j16 in range(8):
                            vs.append(
                                plsc.load_gather(gv, [rows[j16], cold])
                            )
                    for k, d in enumerate((d0, d0 + 1)):
                        for j16 in range(8):
                            tv[
                                blk, d // 8, d % 8, pl.ds(j16 * 16, 16)
                            ] = vs[k * 8 + j16]

            for b in range(8):
                pltpu.async_copy(
                    tv.at[b], out_hbm.at[8 * g + b, :, bt0 + btl], osem
                )

        pltpu.sync_copy(idx_hbm.at[0, pl.ds(bt0, BT_PER_W)], idx_v.at[0])
        fire_gathers(0)
        fire_slab(1, 1)

        def body(gi, carry):
            s = gi + 1

            @pl.when((s & 3) == 0)
            def _():
                drain_slab()

            fire_gathers(s)
            drain_gathers()

            @pl.when(((s & 3) == 0) & (s <= 4 * (HG - 1) - 4))
            def _():
                fire_slab((s >> 2) + 1, ((s >> 2) + 1) & 1)

            @pl.when(gi > 0)
            def _():
                drain_stores()

            process(gi)
            return carry

        lax.fori_loop(0, NSLOT - 1, body, 0)
        drain_stores()
        drain_gathers()
        process(NSLOT - 1)
        drain_stores()

    return lookup


_LOOKUP = _make_lookup()


def kernel(inputs, embeddings):
    idx_p = inputs.T.reshape(HG, 8, BT, 128).transpose(0, 2, 1, 3)
    out_p = _LOOKUP(idx_p, embeddings)
    x = out_p.transpose(0, 1, 3, 2, 4).reshape(H, D, B)
    return x.transpose(2, 0, 1)


# speedup vs baseline: 7.4729x; 5.8174x over previous
import functools

import jax
import jax.numpy as jnp
from jax import lax
from jax.experimental import pallas as pl
from jax.experimental.pallas import tpu as pltpu
from jax.experimental.pallas import tpu_sc as plsc

D = 32
NC = 2
NS = 16
NW = NC * NS
B = 16384
H = 200
BT = B // 128
BT_PER_W = BT // NW
HG = H // 8
NSLOT = HG * BT_PER_W


def _make_lookup():
    mesh = plsc.VectorSubcoreMesh(core_axis_name="c", subcore_axis_name="s")

    @functools.partial(
        pl.kernel,
        out_type=jax.ShapeDtypeStruct((H, D // 8, BT, 8, 128), jnp.float32),
        mesh=mesh,
        scratch_types=[
            pltpu.VMEM((2, BT_PER_W, 8, 128), jnp.int32),
            pltpu.VMEM((2048, D), jnp.float32),
            pltpu.VMEM((8, D // 8, 8, 128), jnp.float32),
            pltpu.SemaphoreType.DMA,
            pltpu.SemaphoreType.DMA,
            pltpu.SemaphoreType.DMA,
        ],
        compiler_params=pltpu.CompilerParams(
            use_tc_tiling_on_sc=False, needs_layout_passes=False
        ),
    )
    def lookup(idx_hbm, table_hbm, out_hbm, idx_v, gv, tv, gsem, osem, isem):
        wid = lax.axis_index("s") * NC + lax.axis_index("c")
        bt0 = wid * BT_PER_W
        iota = jnp.arange(16, dtype=jnp.int32)

        def fire_slab(g, q):
            pltpu.async_copy(
                idx_hbm.at[g, pl.ds(bt0, BT_PER_W)], idx_v.at[q], isem
            )

        def drain_slab():
            pltpu.make_async_copy(
                idx_hbm.at[0, pl.ds(0, BT_PER_W)], idx_v.at[0], isem
            ).wait()

        def fire_gathers(s):
            g = s >> 2
            btl = s & 3
            q = g & 1
            p = s & 1
            for i in range(8):
                pltpu.async_copy(
                    table_hbm.at[idx_v.at[q, btl, i]],
                    gv.at[pl.ds(p * 1024 + i * 128, 128)],
                    gsem,
                )

        def drain_gathers():
            for i in range(8):
                pltpu.make_async_copy(
                    table_hbm.at[pl.ds(0, 128)],
                    gv.at[pl.ds(i * 128, 128)],
                    gsem,
                ).wait()

        def drain_stores():
            for b in range(8):
                pltpu.make_async_copy(
                    tv.at[b], out_hbm.at[0, :, 0], osem
                ).wait()

        def process(s):
            g = s >> 2
            btl = s & 3
            p = s & 1
            base0 = p * 1024

            zero16 = jnp.zeros((16,), dtype=jnp.int32)

            @plsc.parallel_loop(0, 8, unroll=2)
            def tblk(blk):
                base = base0 + blk * 128
                blkv = zero16 + blk
                rows = [iota + (base + j16 * 16) for j16 in range(8)]
                jvs = [iota + j16 * 16 for j16 in range(8)]

                @plsc.parallel_loop(0, 16)
                def kbody(k):
                    dvec = (iota + k) & 15
                    for dh in range(2):
                        col = dvec + dh * 16
                        dtv = (dvec >> 3) + dh * 2
                        div = dvec & 7
                        vs = [
                            plsc.load_gather(gv, [rows[j16], col])
                            for j16 in range(8)
                        ]
                        for j16 in range(8):
                            plsc.store_scatter(
                                tv, [blkv, dtv, div, jvs[j16]], vs[j16]
                            )

            for b in range(8):
                pltpu.async_copy(
                    tv.at[b], out_hbm.at[8 * g + b, :, bt0 + btl], osem
                )

        pltpu.sync_copy(idx_hbm.at[0, pl.ds(bt0, BT_PER_W)], idx_v.at[0])
        fire_gathers(0)
        fire_slab(1, 1)

        def body(gi, carry):
            s = gi + 1

            @pl.when((s & 3) == 0)
            def _():
                drain_slab()

            fire_gathers(s)
            drain_gathers()

            @pl.when(((s & 3) == 0) & (s <= 4 * (HG - 1) - 4))
            def _():
                fire_slab((s >> 2) + 1, ((s >> 2) + 1) & 1)

            @pl.when(gi > 0)
            def _():
                drain_stores()

            process(gi)
            return carry

        lax.fori_loop(0, NSLOT - 1, body, 0)
        drain_stores()
        drain_gathers()
        process(NSLOT - 1)
        drain_stores()

    return lookup


_LOOKUP = _make_lookup()


def kernel(inputs, embeddings):
    idx_p = inputs.T.reshape(HG, 8, BT, 128).transpose(0, 2, 1, 3)
    out_p = _LOOKUP(idx_p, embeddings)
    x = out_p.transpose(0, 1, 3, 2, 4).reshape(H, D, B)
    return x.transpose(2, 0, 1)
